# Initial kernel scaffold; baseline (speedup 1.0000x reference)
import functools

import jax
import jax.numpy as jnp
from jax import lax
from jax.experimental import pallas as pl
from jax.experimental.pallas import tpu as pltpu
from jax.experimental.pallas import tpu_sc as plsc

_LANES = 16
_NUM_CORES = 2
_NUM_SUBCORES = 16
_NUM_WORKERS = _NUM_CORES * _NUM_SUBCORES
_UNROLL = 8


def _sc_pool(x_flat, emb, batch, seq, dim):
    nw = _NUM_WORKERS
    bpw = batch // nw
    ipw = bpw * seq

    chunks = []
    off = 0
    while off < seq:
        c = min(128, seq - off)
        chunks.append((off, c))
        off += c

    nvec = dim // _LANES
    mesh = plsc.VectorSubcoreMesh(core_axis_name="c", subcore_axis_name="s")

    @functools.partial(
        pl.kernel,
        mesh=mesh,
        out_type=jax.ShapeDtypeStruct((batch, dim), jnp.float32),
        scratch_types=[
            pltpu.VMEM((ipw,), jnp.int32),
            pltpu.VMEM((seq, dim), jnp.float32),
            pltpu.VMEM((bpw, dim), jnp.float32),
        ],
    )
    def pool(x_hbm, emb_hbm, out_hbm, idx_v, rows_v, pooled_v):
        wid = lax.axis_index("s") * _NUM_CORES + lax.axis_index("c")
        base = pl.multiple_of(wid * ipw, 8)
        pltpu.sync_copy(x_hbm.at[pl.ds(base, ipw)], idx_v)

        def do_elem(e, carry):
            for off, c in chunks:
                start = pl.multiple_of(e * seq + off, 8)
                pltpu.sync_copy(
                    emb_hbm.at[idx_v.at[pl.ds(start, c)]],
                    rows_v.at[pl.ds(off, c)],
                )

            def acc_body(i, accs):
                l0 = i * _UNROLL
                new = list(accs)
                for dl in range(_UNROLL):
                    for v in range(nvec):
                        new[v] = new[v] + rows_v[l0 + dl, pl.ds(v * _LANES, _LANES)]
                return tuple(new)

            zero = jnp.zeros((_LANES,), jnp.float32)
            accs = lax.fori_loop(0, seq // _UNROLL, acc_body, (zero,) * nvec)
            for l in range(seq - seq % _UNROLL, seq):
                accs = tuple(
                    accs[v] + rows_v[l, pl.ds(v * _LANES, _LANES)]
                    for v in range(nvec)
                )
            for v in range(nvec):
                pooled_v[e, pl.ds(v * _LANES, _LANES)] = accs[v]
            return carry

        lax.fori_loop(0, bpw, do_elem, 0)
        pltpu.sync_copy(pooled_v, out_hbm.at[pl.ds(wid * bpw, bpw)])

    return pool(x_flat, emb)


def _mlp_head(pooled, W1, b1, W2, b2, seq):
    batch, dim = pooled.shape
    hidden = W1.shape[1]

    def body(p_ref, w1_ref, b1_ref, w2_ref, b2_ref, o_ref):
        h = p_ref[...] * (1.0 / seq)
        z = jnp.dot(h, w1_ref[...], preferred_element_type=jnp.float32)
        z = jnp.maximum(z + b1_ref[...], 0.0)
        logit = jnp.dot(z, w2_ref[...], preferred_element_type=jnp.float32)
        o_ref[...] = jax.nn.sigmoid(logit + b2_ref[...])

    out = pl.pallas_call(
        body,
        out_shape=jax.ShapeDtypeStruct((batch, 1), jnp.float32),
    )(pooled, W1, b1.reshape(1, hidden), W2, b2.reshape(1, 1))
    return out.reshape(batch)


def kernel(x, emb, W1, b1, W2, b2):
    batch, seq = x.shape
    _, dim = emb.shape
    pooled = _sc_pool(x.reshape(-1), emb, batch, seq, dim)
    return _mlp_head(pooled, W1, b1, W2, b2, seq)


# speedup vs baseline: 7.6364x; 7.6364x over previous
import functools

import jax
import jax.numpy as jnp
from jax import lax
from jax.experimental import pallas as pl
from jax.experimental.pallas import tpu as pltpu
from jax.experimental.pallas import tpu_sc as plsc

_LANES = 16
_NUM_CORES = 2
_NUM_SUBCORES = 16
_NUM_WORKERS = _NUM_CORES * _NUM_SUBCORES
_UNROLL = 8


def _sc_pool(x_flat, emb, batch, seq, dim):
    nw = _NUM_WORKERS
    bpw = batch // nw
    ipw = bpw * seq

    chunks = []
    off = 0
    while off < seq:
        c = min(128, seq - off)
        chunks.append((off, c))
        off += c

    nvec = dim // _LANES
    mesh = plsc.VectorSubcoreMesh(core_axis_name="c", subcore_axis_name="s")

    @functools.partial(
        pl.kernel,
        mesh=mesh,
        compiler_params=pltpu.CompilerParams(use_tc_tiling_on_sc=False),
        out_type=jax.ShapeDtypeStruct((batch, dim), jnp.float32),
        scratch_types=[
            pltpu.VMEM((ipw,), jnp.int32),
            pltpu.VMEM((seq, dim), jnp.float32),
            pltpu.VMEM((bpw, dim), jnp.float32),
        ],
    )
    def pool(x_hbm, emb_hbm, out_hbm, idx_v, rows_v, pooled_v):
        wid = lax.axis_index("s") * _NUM_CORES + lax.axis_index("c")
        base = pl.multiple_of(wid * ipw, 8)
        pltpu.sync_copy(x_hbm.at[pl.ds(base, ipw)], idx_v)

        def do_elem(e, carry):
            for off, c in chunks:
                start = pl.multiple_of(e * seq + off, 8)
                pltpu.sync_copy(
                    emb_hbm.at[idx_v.at[pl.ds(start, c)]],
                    rows_v.at[pl.ds(off, c)],
                )

            def acc_body(i, accs):
                l0 = i * _UNROLL
                new = list(accs)
                for dl in range(_UNROLL):
                    for v in range(nvec):
                        new[v] = new[v] + rows_v[l0 + dl, pl.ds(v * _LANES, _LANES)]
                return tuple(new)

            zero = jnp.zeros((_LANES,), jnp.float32)
            accs = lax.fori_loop(0, seq // _UNROLL, acc_body, (zero,) * nvec)
            for l in range(seq - seq % _UNROLL, seq):
                accs = tuple(
                    accs[v] + rows_v[l, pl.ds(v * _LANES, _LANES)]
                    for v in range(nvec)
                )
            for v in range(nvec):
                pooled_v[e, pl.ds(v * _LANES, _LANES)] = accs[v]
            return carry

        lax.fori_loop(0, bpw, do_elem, 0)
        pltpu.sync_copy(pooled_v, out_hbm.at[pl.ds(wid * bpw, bpw)])

    return pool(x_flat, emb)


def _mlp_head(pooled, W1, b1, W2, b2, seq):
    batch, dim = pooled.shape
    hidden = W1.shape[1]

    def body(p_ref, w1_ref, b1_ref, w2_ref, b2_ref, o_ref):
        h = p_ref[...] * (1.0 / seq)
        z = jnp.dot(h, w1_ref[...], preferred_element_type=jnp.float32)
        z = jnp.maximum(z + b1_ref[...], 0.0)
        logit = jnp.dot(z, w2_ref[...], preferred_element_type=jnp.float32)
        o_ref[...] = jax.nn.sigmoid(logit + b2_ref[...])

    out = pl.pallas_call(
        body,
        out_shape=jax.ShapeDtypeStruct((batch, 1), jnp.float32),
    )(pooled, W1, b1.reshape(1, hidden), W2, b2.reshape(1, 1))
    return out.reshape(batch)


def kernel(x, emb, W1, b1, W2, b2):
    batch, seq = x.shape
    _, dim = emb.shape
    pooled = _sc_pool(x.reshape(-1), emb, batch, seq, dim)
    return _mlp_head(pooled, W1, b1, W2, b2, seq)


# speedup vs baseline: 17.4602x; 2.2865x over previous
import functools

import jax
import jax.numpy as jnp
from jax import lax
from jax.experimental import pallas as pl
from jax.experimental.pallas import tpu as pltpu
from jax.experimental.pallas import tpu_sc as plsc

_LANES = 16
_NUM_CORES = 2
_NUM_SUBCORES = 16
_NUM_WORKERS = _NUM_CORES * _NUM_SUBCORES
_UNROLL = 20
_NBUF = 4


def _sc_pool(x_flat, emb, batch, seq, dim):
    nw = _NUM_WORKERS
    bpw = batch // nw
    ipw = bpw * seq

    chunks = []
    off = 0
    while off < seq:
        c = min(128, seq - off)
        chunks.append((off, c))
        off += c

    nvec = dim // _LANES
    mesh = plsc.VectorSubcoreMesh(core_axis_name="c", subcore_axis_name="s")

    @functools.partial(
        pl.kernel,
        mesh=mesh,
        compiler_params=pltpu.CompilerParams(use_tc_tiling_on_sc=False),
        out_type=jax.ShapeDtypeStruct((batch, dim), jnp.float32),
        scratch_types=[
            pltpu.VMEM((ipw,), jnp.int32),
            pltpu.VMEM((_NBUF, seq, dim), jnp.float32),
            pltpu.VMEM((bpw, dim), jnp.float32),
        ] + [pltpu.SemaphoreType.DMA] * _NBUF,
    )
    def pool(x_hbm, emb_hbm, out_hbm, idx_v, rows_v, pooled_v, *sems):
        wid = lax.axis_index("s") * _NUM_CORES + lax.axis_index("c")
        base = pl.multiple_of(wid * ipw, 8)
        pltpu.sync_copy(x_hbm.at[pl.ds(base, ipw)], idx_v)

        def issue(e, b):
            for off, c in chunks:
                start = pl.multiple_of(e * seq + off, 8)
                pltpu.async_copy(
                    emb_hbm.at[idx_v.at[pl.ds(start, c)]],
                    rows_v.at[b].at[pl.ds(off, c)],
                    sems[b],
                )

        def drain(e, b):
            for off, c in chunks:
                start = pl.multiple_of(e * seq + off, 8)
                pltpu.make_async_copy(
                    emb_hbm.at[idx_v.at[pl.ds(start, c)]],
                    rows_v.at[b].at[pl.ds(off, c)],
                    sems[b],
                ).wait()

        for b in range(_NBUF):
            issue(b, b)

        def do_group(g, carry):
            for b in range(_NBUF):
                e = g * _NBUF + b
                drain(e, b)

                def acc_body(i, accs):
                    l0 = i * _UNROLL
                    new = list(accs)
                    for dl in range(_UNROLL):
                        for v in range(nvec):
                            new[v] = new[v] + rows_v[b, l0 + dl, pl.ds(v * _LANES, _LANES)]
                    return tuple(new)

                zero = jnp.zeros((_LANES,), jnp.float32)
                accs = lax.fori_loop(0, seq // _UNROLL, acc_body, (zero,) * nvec)
                for l in range(seq - seq % _UNROLL, seq):
                    accs = tuple(
                        accs[v] + rows_v[b, l, pl.ds(v * _LANES, _LANES)]
                        for v in range(nvec)
                    )
                for v in range(nvec):
                    pooled_v[e, pl.ds(v * _LANES, _LANES)] = accs[v]

                @pl.when(e + _NBUF < bpw)
                def _():
                    issue(e + _NBUF, b)

            return carry

        lax.fori_loop(0, bpw // _NBUF, do_group, 0)
        pltpu.sync_copy(pooled_v, out_hbm.at[pl.ds(wid * bpw, bpw)])

    return pool(x_flat, emb)


def _mlp_head(pooled, W1, b1, W2, b2, seq):
    batch, dim = pooled.shape
    hidden = W1.shape[1]

    def body(p_ref, w1_ref, b1_ref, w2_ref, b2_ref, o_ref):
        h = p_ref[...] * (1.0 / seq)
        z = jnp.dot(h, w1_ref[...], preferred_element_type=jnp.float32)
        z = jnp.maximum(z + b1_ref[...], 0.0)
        logit = jnp.dot(z, w2_ref[...], preferred_element_type=jnp.float32)
        o_ref[...] = jax.nn.sigmoid(logit + b2_ref[...])

    out = pl.pallas_call(
        body,
        out_shape=jax.ShapeDtypeStruct((batch, 1), jnp.float32),
    )(pooled, W1, b1.reshape(1, hidden), W2, b2.reshape(1, 1))
    return out.reshape(batch)


def kernel(x, emb, W1, b1, W2, b2):
    batch, seq = x.shape
    _, dim = emb.shape
    pooled = _sc_pool(x.reshape(-1), emb, batch, seq, dim)
    return _mlp_head(pooled, W1, b1, W2, b2, seq)
